# Initial kernel scaffold; baseline (speedup 1.0000x reference)
import jax
import jax.numpy as jnp
from jax.experimental import pallas as pl


def kernel(x, w1, b1, w2, b2, dw1, db1, dw2, db2, emb):
    raise NotImplementedError("write your pallas kernel here")


# speedup vs baseline: 3.0492x; 3.0492x over previous
import functools

import jax
import jax.numpy as jnp
from jax import lax
from jax.experimental import pallas as pl

F32 = jnp.float32
B_TOTAL = 256
BN = 128


def _mm(a, b):
    return lax.dot_general(a, b, (((1,), (0,)), ((), ())),
                           preferred_element_type=F32)


def _pad_hw(a):
    c, h, w, n = a.shape
    zc = jnp.zeros((c, h, 1, n), a.dtype)
    a = jnp.concatenate([zc, a, zc], axis=2)
    zr = jnp.zeros((c, 1, w + 2, n), a.dtype)
    return jnp.concatenate([zr, a, zr], axis=1)


def _tap2(ap, ky, kx, ho, wo):
    c, _, _, n = ap.shape
    win = ap[:, ky:ky + 2 * ho, kx:kx + 2 * wo, :]
    win = win.reshape(c, ho, 2, 2 * wo, n)[:, :, 0, :, :]
    win = win.reshape(c, ho, wo, 2, n)[:, :, :, 0, :]
    return win


def _interleave2(p00, p01, p10, p11):
    c, h, w, n = p00.shape
    r0 = jnp.concatenate([p00[:, :, :, None, :], p01[:, :, :, None, :]],
                         axis=3).reshape(c, h, 2 * w, n)
    r1 = jnp.concatenate([p10[:, :, :, None, :], p11[:, :, :, None, :]],
                         axis=3).reshape(c, h, 2 * w, n)
    return jnp.concatenate([r0[:, :, None, :, :], r1[:, :, None, :, :]],
                           axis=2).reshape(c, 2 * h, 2 * w, n)


def _vqvae_body(xp_ref, w1m_ref, w2m_ref, dw1m_ref, dw2m_ref,
                b1_ref, b2_ref, db1_ref, db2_ref, emb_ref, embt_ref,
                out_ref):
    n = BN
    xp = xp_ref[...]

    acc1 = jnp.zeros((32, 16 * 16 * n), F32)
    for t, (ky, kx) in enumerate([(i, j) for i in range(3) for j in range(3)]):
        tap = _tap2(xp, ky, kx, 16, 16).reshape(3, 16 * 16 * n)
        acc1 = acc1 + _mm(w1m_ref[t * 32:(t + 1) * 32, :], tap)
    h1 = jnp.maximum(acc1 + b1_ref[...], 0.0).reshape(32, 16, 16, n)

    h1p = _pad_hw(h1)
    acc2 = jnp.zeros((64, 8 * 8 * n), F32)
    for t, (ky, kx) in enumerate([(i, j) for i in range(3) for j in range(3)]):
        tap = _tap2(h1p, ky, kx, 8, 8).reshape(32, 8 * 8 * n)
        acc2 = acc2 + _mm(w2m_ref[t * 64:(t + 1) * 64, :], tap)
    xenc = acc2 + b2_ref[...]

    emb = emb_ref[...]
    embt = embt_ref[...]
    embsq = jnp.sum(emb * emb, axis=1)[:, None]
    s = embsq - 2.0 * _mm(emb, xenc)
    smin = jnp.min(s, axis=0)[None, :]
    codei = lax.broadcasted_iota(jnp.int32, s.shape, 0)
    cand = jnp.where(s == smin, codei, 64)
    tok = jnp.min(cand, axis=0)[None, :]
    onehot = (codei == tok).astype(F32)
    e = _mm(embt, onehot)

    dvq = e - xenc
    vql = jnp.sum(jnp.sqrt(jnp.sum(dvq * dvq, axis=0)))

    vqp = _pad_hw(e.reshape(64, 8, 8, n))
    ps = []
    for p, (py, px) in enumerate([(i, j) for i in range(2) for j in range(2)]):
        accp = jnp.zeros((32, 8 * 8 * n), F32)
        for t, (a, b) in enumerate([(i, j) for i in range(2) for j in range(2)]):
            win = vqp[:, py + a:py + a + 8, px + b:px + b + 8, :]
            r = (p * 4 + t) * 32
            accp = accp + _mm(dw1m_ref[r:r + 32, :], win.reshape(64, 8 * 8 * n))
        ps.append(jnp.maximum(accp + db1_ref[...], 0.0).reshape(32, 8, 8, n))
    d = _interleave2(*ps)

    dp = _pad_hw(d)
    xi = xp[:, 1:33, 1:33, :]
    rec = jnp.zeros((), F32)
    for p, (qy, qx) in enumerate([(i, j) for i in range(2) for j in range(2)]):
        accq = jnp.zeros((3, 16 * 16 * n), F32)
        for t, (a, b) in enumerate([(i, j) for i in range(2) for j in range(2)]):
            win = dp[:, qy + a:qy + a + 16, qx + b:qx + b + 16, :]
            r = (p * 4 + t) * 3
            accq = accq + _mm(dw2m_ref[r:r + 3, :], win.reshape(32, 16 * 16 * n))
        q = accq + db2_ref[...]
        xq = xi.reshape(3, 16, 2, 32, n)[:, :, qy, :, :]
        xq = xq.reshape(3, 16, 16, 2, n)[:, :, :, qx, :].reshape(3, 16 * 16 * n)
        dq = xq - q
        rec = rec + jnp.sum(dq * dq)

    block_loss = (0.5 * rec + 1.1 * vql) * (1.0 / B_TOTAL)

    @pl.when(pl.program_id(0) == 0)
    def _():
        out_ref[...] = jnp.zeros_like(out_ref)

    out_ref[...] = out_ref[...] + block_loss


def kernel(x, w1, b1, w2, b2, dw1, db1, dw2, db2, emb):
    xp = jnp.pad(jnp.transpose(x, (1, 2, 3, 0)),
                 ((0, 0), (1, 1), (1, 1), (0, 0)))
    w1m = jnp.transpose(w1, (2, 3, 0, 1)).reshape(9 * 32, 3)
    w2m = jnp.transpose(w2, (2, 3, 0, 1)).reshape(9 * 64, 32)
    dw1m = jnp.concatenate(
        [dw1[:, :, 3 - 2 * a - py, 3 - 2 * b - px].T
         for py in (0, 1) for px in (0, 1)
         for a in (0, 1) for b in (0, 1)], axis=0)
    dw2m = jnp.concatenate(
        [dw2[:, :, 3 - 2 * a - py, 3 - 2 * b - px].T
         for py in (0, 1) for px in (0, 1)
         for a in (0, 1) for b in (0, 1)], axis=0)
    b1c = b1[:, None]
    b2c = b2[:, None]
    db1c = db1[:, None]
    db2c = db2[:, None]
    embt = emb.T

    grid = (B_TOTAL // BN,)
    full = lambda r: pl.BlockSpec(None, lambda i: (0,) * r)
    out = pl.pallas_call(
        _vqvae_body,
        grid=grid,
        in_specs=[
            pl.BlockSpec((3, 34, 34, BN), lambda i: (0, 0, 0, i)),
            full(2), full(2), full(2), full(2),
            full(2), full(2), full(2), full(2),
            full(2), full(2),
        ],
        out_specs=pl.BlockSpec((1, 1), lambda i: (0, 0)),
        out_shape=jax.ShapeDtypeStruct((1, 1), F32),
    )(xp, w1m, w2m, dw1m, dw2m, b1c, b2c, db1c, db2c, emb, embt)
    return out[0, 0]


# speedup vs baseline: 3.0493x; 1.0000x over previous
import jax
import jax.numpy as jnp
from jax import lax
from jax.experimental import pallas as pl

F32 = jnp.float32
B_TOTAL = 256
BN = 128


def _mm(a, b):
    return lax.dot_general(a, b, (((1,), (0,)), ((), ())),
                           preferred_element_type=F32)


def _pad_hw(a):
    c, h, w, n = a.shape
    zc = jnp.zeros((c, h, 1, n), a.dtype)
    a = jnp.concatenate([zc, a, zc], axis=2)
    zr = jnp.zeros((c, 1, w + 2, n), a.dtype)
    return jnp.concatenate([zr, a, zr], axis=1)


def _tap2(ap, ky, kx, ho, wo):
    c, _, _, n = ap.shape
    win = ap[:, ky:ky + 2 * ho, kx:kx + 2 * wo, :]
    win = win.reshape(c, ho, 2, 2 * wo, n)[:, :, 0, :, :]
    win = win.reshape(c, ho, wo, 2, n)[:, :, :, 0, :]
    return win


def _interleave2(p00, p01, p10, p11):
    c, h, w, n = p00.shape
    r0 = jnp.concatenate([p00[:, :, :, None, :], p01[:, :, :, None, :]],
                         axis=3).reshape(c, h, 2 * w, n)
    r1 = jnp.concatenate([p10[:, :, :, None, :], p11[:, :, :, None, :]],
                         axis=3).reshape(c, h, 2 * w, n)
    return jnp.concatenate([r0[:, :, None, :, :], r1[:, :, None, :, :]],
                           axis=2).reshape(c, 2 * h, 2 * w, n)


def _vqvae_body(xp_ref, w1m_ref, w2m_ref, dw1m_ref, dw2m_ref,
                b1_ref, b2_ref, db1_ref, db2_ref, emb_ref, embt_ref,
                out_ref):
    n = BN
    xp = xp_ref[...]

    acc1 = jnp.zeros((32, 16 * 16 * n), F32)
    for t, (ky, kx) in enumerate([(i, j) for i in range(3) for j in range(3)]):
        tap = _tap2(xp, ky, kx, 16, 16).reshape(3, 16 * 16 * n)
        acc1 = acc1 + _mm(w1m_ref[t * 32:(t + 1) * 32, :], tap)
    h1 = jnp.maximum(acc1 + b1_ref[...], 0.0).reshape(32, 16, 16, n)

    h1p = _pad_hw(h1)
    acc2 = jnp.zeros((64, 8 * 8 * n), F32)
    for t, (ky, kx) in enumerate([(i, j) for i in range(3) for j in range(3)]):
        tap = _tap2(h1p, ky, kx, 8, 8).reshape(32, 8 * 8 * n)
        acc2 = acc2 + _mm(w2m_ref[t * 64:(t + 1) * 64, :], tap)
    xenc = acc2 + b2_ref[...]

    emb = emb_ref[...]
    embt = embt_ref[...]
    embsq = jnp.sum(emb * emb, axis=1)[:, None]
    s = embsq - 2.0 * _mm(emb, xenc)
    smin = jnp.min(s, axis=0)[None, :]
    codei = lax.broadcasted_iota(jnp.int32, s.shape, 0)
    cand = jnp.where(s == smin, codei, 64)
    tok = jnp.min(cand, axis=0)[None, :]
    onehot = (codei == tok).astype(F32)
    e = _mm(embt, onehot)

    tsq = jnp.sum(xenc * xenc, axis=0)[None, :]
    vql = jnp.sum(jnp.sqrt(jnp.maximum(smin + tsq, 0.0)))

    vqp = _pad_hw(e.reshape(64, 8, 8, n))
    ps = []
    for p, (py, px) in enumerate([(i, j) for i in range(2) for j in range(2)]):
        accp = jnp.zeros((32, 8 * 8 * n), F32)
        for t, (a, b) in enumerate([(i, j) for i in range(2) for j in range(2)]):
            win = vqp[:, py + a:py + a + 8, px + b:px + b + 8, :]
            r = (p * 4 + t) * 32
            accp = accp + _mm(dw1m_ref[r:r + 32, :], win.reshape(64, 8 * 8 * n))
        ps.append(jnp.maximum(accp + db1_ref[...], 0.0).reshape(32, 8, 8, n))
    d = _interleave2(*ps)

    dp = _pad_hw(d)
    xi = xp[:, 1:33, 1:33, :]
    rec = jnp.zeros((), F32)
    for p, (qy, qx) in enumerate([(i, j) for i in range(2) for j in range(2)]):
        accq = jnp.zeros((3, 16 * 16 * n), F32)
        for t, (a, b) in enumerate([(i, j) for i in range(2) for j in range(2)]):
            win = dp[:, qy + a:qy + a + 16, qx + b:qx + b + 16, :]
            r = (p * 4 + t) * 3
            accq = accq + _mm(dw2m_ref[r:r + 3, :], win.reshape(32, 16 * 16 * n))
        q = accq + db2_ref[...]
        xq = xi.reshape(3, 16, 2, 32, n)[:, :, qy, :, :]
        xq = xq.reshape(3, 16, 16, 2, n)[:, :, :, qx, :].reshape(3, 16 * 16 * n)
        dq = xq - q
        rec = rec + jnp.sum(dq * dq)

    block_loss = (0.5 * rec + 1.1 * vql) * (1.0 / B_TOTAL)

    @pl.when(pl.program_id(0) == 0)
    def _():
        out_ref[...] = jnp.zeros_like(out_ref)

    out_ref[...] = out_ref[...] + block_loss


def kernel(x, w1, b1, w2, b2, dw1, db1, dw2, db2, emb):
    xp = jnp.pad(jnp.transpose(x, (1, 2, 3, 0)),
                 ((0, 0), (1, 1), (1, 1), (0, 0)))
    w1m = jnp.transpose(w1, (2, 3, 0, 1)).reshape(9 * 32, 3)
    w2m = jnp.transpose(w2, (2, 3, 0, 1)).reshape(9 * 64, 32)
    dw1m = jnp.concatenate(
        [dw1[:, :, 3 - 2 * a - py, 3 - 2 * b - px].T
         for py in (0, 1) for px in (0, 1)
         for a in (0, 1) for b in (0, 1)], axis=0)
    dw2m = jnp.concatenate(
        [dw2[:, :, 3 - 2 * a - py, 3 - 2 * b - px].T
         for py in (0, 1) for px in (0, 1)
         for a in (0, 1) for b in (0, 1)], axis=0)
    b1c = b1[:, None]
    b2c = b2[:, None]
    db1c = db1[:, None]
    db2c = db2[:, None]
    embt = emb.T

    grid = (B_TOTAL // BN,)
    full = lambda r: pl.BlockSpec(None, lambda i: (0,) * r)
    out = pl.pallas_call(
        _vqvae_body,
        grid=grid,
        in_specs=[
            pl.BlockSpec((3, 34, 34, BN), lambda i: (0, 0, 0, i)),
            full(2), full(2), full(2), full(2),
            full(2), full(2), full(2), full(2),
            full(2), full(2),
        ],
        out_specs=pl.BlockSpec((1, 1), lambda i: (0, 0)),
        out_shape=jax.ShapeDtypeStruct((1, 1), F32),
    )(xp, w1m, w2m, dw1m, dw2m, b1c, b2c, db1c, db2c, emb, embt)
    return out[0, 0]
